# Initial kernel scaffold; baseline (speedup 1.0000x reference)
import functools

import jax
import jax.numpy as jnp
from jax import lax
from jax.experimental import pallas as pl
from jax.experimental.pallas import tpu as pltpu
from jax.experimental.pallas import tpu_sc as plsc

N = 100000
E = 1600000
G = 128

NC, NS, LANES = 2, 16, 16
SUB = 128
UCH = 8
EPAD = 1638400
ESUB = EPAD // SUB
NSINK = 16
NZ = N + NSINK
NPT = N // NS

_MESH = plsc.VectorSubcoreMesh(
    core_axis_name="c", subcore_axis_name="s", num_cores=NC, num_subcores=NS)


@functools.partial(
    pl.kernel,
    out_type=jax.ShapeDtypeStruct((2 * N, 1), jnp.float32),
    mesh=_MESH,
    scratch_types=[
        pltpu.VMEM((UCH, SUB), jnp.int32),
        pltpu.VMEM((SUB, 1), jnp.float32),
        pltpu.VMEM_SHARED((NZ, 1), jnp.float32),
        pltpu.SemaphoreType.DMA,
    ],
)
def _deg_kernel(dstr_hbm, zeros1_hbm, ones_hbm, out_hbm,
                dst_v, ones_v, zacc, ssem):
    c = lax.axis_index("c")
    s = lax.axis_index("s")
    pltpu.sync_copy(ones_hbm, ones_v)
    pltpu.sync_copy(zeros1_hbm.at[pl.ds(s * NPT, NPT)],
                    zacc.at[pl.ds(s * NPT, NPT)])
    plsc.subcore_barrier()
    sub_per_tile = (ESUB // 2) // NS
    tile_base = c * (ESUB // 2) + s * sub_per_tile

    def body(k, carry):
        base = tile_base + k * UCH
        pltpu.sync_copy(dstr_hbm.at[pl.ds(base, UCH)], dst_v)
        descs = [
            pltpu.async_copy(ones_v, zacc.at[dst_v.at[j]], ssem, add=True)
            for j in range(UCH)
        ]
        for d in descs:
            d.wait()
        return carry

    lax.fori_loop(0, sub_per_tile // UCH, body, 0)
    plsc.subcore_barrier()
    pltpu.sync_copy(zacc.at[pl.ds(s * NPT, NPT)],
                    out_hbm.at[pl.ds(c * N + s * NPT, NPT)])


def _make_agg(nparts, edge_split):
    tasks = 1 if edge_split else nparts // NC
    nslots = 2 if edge_split else nparts
    sub_per_task = (ESUB // 2) if edge_split else ESUB
    sub_per_tile = sub_per_task // NS
    n_outer = sub_per_tile // UCH

    @functools.partial(
        pl.kernel,
        out_type=jax.ShapeDtypeStruct((nslots * N, LANES), jnp.float32),
        mesh=_MESH,
        scratch_types=[
            pltpu.VMEM((UCH, SUB), jnp.int32),
            pltpu.VMEM((UCH, SUB), jnp.int32),
            pltpu.VMEM((UCH, SUB, LANES), jnp.float32),
            pltpu.VMEM_SHARED((NZ, LANES), jnp.float32),
            pltpu.SemaphoreType.DMA,
            pltpu.SemaphoreType.DMA,
        ],
    )
    def agg(y_hbm, srcr_hbm, dstr_hbm, zeros_hbm, out_hbm,
            src_v, dst_v, rows_v, zacc, gsem, ssem):
        c = lax.axis_index("c")
        s = lax.axis_index("s")
        for q in range(tasks):
            if edge_split:
                part_off = None
                slot = c
                task_base = c * sub_per_task
            else:
                part = c * tasks + q
                part_off = part * N
                slot = part
                task_base = 0
            pltpu.sync_copy(zeros_hbm.at[pl.ds(s * NPT, NPT)],
                            zacc.at[pl.ds(s * NPT, NPT)])
            plsc.subcore_barrier()
            tile_base = task_base + s * sub_per_tile

            def body(k, carry, part_off=part_off, tile_base=tile_base):
                base = tile_base + k * UCH
                pltpu.sync_copy(srcr_hbm.at[pl.ds(base, UCH)], src_v)
                pltpu.sync_copy(dstr_hbm.at[pl.ds(base, UCH)], dst_v)
                if part_off is not None:
                    for j in range(UCH):
                        for v in range(SUB // LANES):
                            sl = src_v[j, pl.ds(v * LANES, LANES)]
                            src_v[j, pl.ds(v * LANES, LANES)] = sl + part_off
                gds = [
                    pltpu.async_copy(y_hbm.at[src_v.at[j]], rows_v.at[j], gsem)
                    for j in range(UCH)
                ]
                for d in gds:
                    d.wait()
                sds = [
                    pltpu.async_copy(rows_v.at[j], zacc.at[dst_v.at[j]],
                                     ssem, add=True)
                    for j in range(UCH)
                ]
                for d in sds:
                    d.wait()
                return carry

            lax.fori_loop(0, n_outer, body, 0)
            plsc.subcore_barrier()
            pltpu.sync_copy(zacc.at[pl.ds(s * NPT, NPT)],
                            out_hbm.at[pl.ds(slot * N + s * NPT, NPT)])
            plsc.subcore_barrier()

    return agg


_agg_l1 = _make_agg(1, edge_split=True)
_agg_l2 = _make_agg(2, edge_split=False)
_agg_l3 = _make_agg(4, edge_split=False)


BLK = 2000
NBLK = N // BLK


def _tc0_body(hist_ref, x_ref, w1_ref, dinv_ref, y1_ref):
    deg = hist_ref[0] + hist_ref[1] + 1.0
    dinv = lax.rsqrt(deg)
    dinv_ref[...] = dinv
    xw = jnp.dot(x_ref[...], w1_ref[...], preferred_element_type=jnp.float32)
    y1_ref[...] = xw * dinv


def _tc0(hist, x, w1):
    return pl.pallas_call(
        _tc0_body,
        grid=(NBLK,),
        in_specs=[
            pl.BlockSpec((2, BLK, 1), lambda i: (0, i, 0)),
            pl.BlockSpec((BLK, 5), lambda i: (i, 0)),
            pl.BlockSpec((5, 16), lambda i: (0, 0)),
        ],
        out_specs=[
            pl.BlockSpec((BLK, 1), lambda i: (i, 0)),
            pl.BlockSpec((BLK, 16), lambda i: (i, 0)),
        ],
        out_shape=[
            jax.ShapeDtypeStruct((N, 1), jnp.float32),
            jax.ShapeDtypeStruct((N, 16), jnp.float32),
        ],
    )(hist, x, w1)


def _make_tc_mid(pin, pout, fin, fout, sum_slots):
    def body(z_ref, y_ref, dinv_ref, w_ref, b_ref, out_ref):
        if sum_slots:
            z = z_ref[0] + z_ref[1]
            y = y_ref[0]
        else:
            z = jnp.concatenate([z_ref[p] for p in range(pin)], axis=1)
            y = jnp.concatenate([y_ref[p] for p in range(pin)], axis=1)
        dinv = dinv_ref[...]
        h = jnp.maximum(dinv * (z + y) + b_ref[...], 0.0)
        yn = jnp.dot(h, w_ref[...], preferred_element_type=jnp.float32) * dinv
        for p in range(pout):
            out_ref[p] = yn[:, p * LANES:(p + 1) * LANES]

    zin = 2 if sum_slots else pin
    yin = 1 if sum_slots else pin

    def call(z, y, dinv, w, b):
        return pl.pallas_call(
            body,
            grid=(NBLK,),
            in_specs=[
                pl.BlockSpec((zin, BLK, LANES), lambda i: (0, i, 0)),
                pl.BlockSpec((yin, BLK, LANES), lambda i: (0, i, 0)),
                pl.BlockSpec((BLK, 1), lambda i: (i, 0)),
                pl.BlockSpec((fin, fout), lambda i: (0, 0)),
                pl.BlockSpec((1, fin), lambda i: (0, 0)),
            ],
            out_specs=pl.BlockSpec((pout, BLK, LANES), lambda i: (0, i, 0)),
            out_shape=jax.ShapeDtypeStruct((pout, N, LANES), jnp.float32),
        )(z.reshape(zin, N, LANES), y.reshape(yin, N, LANES), dinv, w,
          b.reshape(1, fin))

    return call


_tc_mid12 = _make_tc_mid(1, 2, 16, 32, sum_slots=True)
_tc_mid23 = _make_tc_mid(2, 4, 32, 64, sum_slots=False)


def _pool_body(z_ref, y_ref, dinv_ref, b_ref, batch_ref, sums_ref, cnts_ref):
    i = pl.program_id(0)
    z = jnp.concatenate([z_ref[p] for p in range(4)], axis=1)
    y = jnp.concatenate([y_ref[p] for p in range(4)], axis=1)
    h = jnp.maximum(dinv_ref[...] * (z + y) + b_ref[...], 0.0)
    bt = batch_ref[...]
    io = lax.broadcasted_iota(jnp.int32, (BLK, G), 1)
    oh = (bt == io).astype(jnp.float32)
    dn = (((0,), (0,)), ((), ()))
    ps = lax.dot_general(oh, h, dn, preferred_element_type=jnp.float32)
    pc = lax.dot_general(oh, jnp.ones((BLK, 1), jnp.float32), dn,
                         preferred_element_type=jnp.float32)

    @pl.when(i == 0)
    def _():
        sums_ref[...] = ps
        cnts_ref[...] = pc

    @pl.when(i != 0)
    def _():
        sums_ref[...] += ps
        cnts_ref[...] += pc


def _tc_pool(z3, y3, dinv, b3, batch2):
    return pl.pallas_call(
        _pool_body,
        grid=(NBLK,),
        in_specs=[
            pl.BlockSpec((4, BLK, LANES), lambda i: (0, i, 0)),
            pl.BlockSpec((4, BLK, LANES), lambda i: (0, i, 0)),
            pl.BlockSpec((BLK, 1), lambda i: (i, 0)),
            pl.BlockSpec((1, 64), lambda i: (0, 0)),
            pl.BlockSpec((BLK, 1), lambda i: (i, 0)),
        ],
        out_specs=[
            pl.BlockSpec((G, 64), lambda i: (0, 0)),
            pl.BlockSpec((G, 1), lambda i: (0, 0)),
        ],
        out_shape=[
            jax.ShapeDtypeStruct((G, 64), jnp.float32),
            jax.ShapeDtypeStruct((G, 1), jnp.float32),
        ],
    )(z3.reshape(4, N, LANES), y3.reshape(4, N, LANES), dinv,
      b3.reshape(1, 64), batch2)


def _head_body(sums_ref, cnts_ref, wfc_ref, bfc_ref, out_ref):
    pooled = sums_ref[...] / jnp.maximum(cnts_ref[...], 1.0)
    logits = jnp.dot(pooled, wfc_ref[...],
                     preferred_element_type=jnp.float32) + bfc_ref[...]
    m = jnp.max(logits, axis=1, keepdims=True)
    e = jnp.exp(logits - m)
    lse = jnp.log(jnp.sum(e, axis=1, keepdims=True)) + m
    out_ref[...] = logits - lse


def _tc_head(sums, cnts, wfc, bfc):
    return pl.pallas_call(
        _head_body,
        out_shape=jax.ShapeDtypeStruct((G, 2), jnp.float32),
    )(sums, cnts, wfc, bfc.reshape(1, 2))


def kernel(x, edge_index, batch, W1, b1, W2, b2, W3, b3, Wfc, bfc):
    src = edge_index[0]
    dst = edge_index[1]
    pad = EPAD - E
    srcp = jnp.concatenate(
        [src, jnp.zeros((pad,), jnp.int32)]).reshape(ESUB, SUB)
    dstp = jnp.concatenate(
        [dst, N + (jnp.arange(pad, dtype=jnp.int32) % NSINK)]
    ).reshape(ESUB, SUB)
    zeros16 = jnp.zeros((N, LANES), jnp.float32)
    zeros1 = jnp.zeros((N, 1), jnp.float32)
    ones1 = jnp.ones((SUB, 1), jnp.float32)

    hist = _deg_kernel(dstp, zeros1, ones1).reshape(2, N, 1)
    dinv, y1 = _tc0(hist, x, W1)
    z1 = _agg_l1(y1, srcp, dstp, zeros16)
    y2 = _tc_mid12(z1, y1, dinv, W2, b1)
    z2 = _agg_l2(y2.reshape(2 * N, LANES), srcp, dstp, zeros16)
    y3 = _tc_mid23(z2, y2, dinv, W3, b2)
    z3 = _agg_l3(y3.reshape(4 * N, LANES), srcp, dstp, zeros16)
    sums, cnts = _tc_pool(z3, y3, dinv, b3, batch.reshape(N, 1))
    return _tc_head(sums, cnts, Wfc, bfc)


# speedup vs baseline: 13.6652x; 13.6652x over previous
import functools

import jax
import jax.numpy as jnp
from jax import lax
from jax.experimental import pallas as pl
from jax.experimental.pallas import tpu as pltpu
from jax.experimental.pallas import tpu_sc as plsc

N = 100000
E = 1600000
G = 128

NC, NS, LANES = 2, 16, 16
SUB = 128
UCH = 8
EPAD = 1638400
ESUB = EPAD // SUB
NP = 100096
NZ = NP
NPT = NP // NS
NSINK = 64

_SC_PARAMS = pltpu.CompilerParams(use_tc_tiling_on_sc=False)


@functools.cache
def _mesh():
    return plsc.VectorSubcoreMesh(
        core_axis_name="c", subcore_axis_name="s",
        num_cores=NC, num_subcores=NS)


@functools.cache
def _make_agg(nparts, edge_split):
    tasks = 1 if edge_split else nparts // NC
    nslots = 2 if edge_split else nparts
    sub_per_task = (ESUB // 2) if edge_split else ESUB
    sub_per_tile = sub_per_task // NS
    n_outer = sub_per_tile // UCH

    @functools.partial(
        pl.kernel,
        out_type=jax.ShapeDtypeStruct((nslots * NP, LANES), jnp.float32),
        mesh=_mesh(),
        compiler_params=_SC_PARAMS,
        scratch_types=[
            pltpu.VMEM((UCH, SUB), jnp.int32),
            pltpu.VMEM((UCH, SUB), jnp.int32),
            pltpu.VMEM((UCH, SUB, LANES), jnp.float32),
            pltpu.VMEM_SHARED((NZ, LANES), jnp.float32),
            pltpu.SemaphoreType.DMA,
            pltpu.SemaphoreType.DMA,
        ],
    )
    def agg(y_hbm, srcr_hbm, dstr_hbm, zeros_hbm, out_hbm,
            src_v, dst_v, rows_v, zacc, gsem, ssem):
        c = lax.axis_index("c")
        s = lax.axis_index("s")
        for q in range(tasks):
            if edge_split:
                part_off = None
                slot = c
                task_base = c * sub_per_task
            else:
                part = c * tasks + q
                part_off = part * N
                slot = part
                task_base = 0
            pltpu.sync_copy(zeros_hbm.at[pl.ds(s * NPT, NPT)],
                            zacc.at[pl.ds(s * NPT, NPT)])
            plsc.subcore_barrier()
            tile_base = task_base + s * sub_per_tile

            def body(k, carry, part_off=part_off, tile_base=tile_base):
                base = tile_base + k * UCH
                pltpu.sync_copy(srcr_hbm.at[pl.ds(base, UCH)], src_v)
                pltpu.sync_copy(dstr_hbm.at[pl.ds(base, UCH)], dst_v)
                if part_off is not None:
                    for j in range(UCH):
                        for v in range(SUB // LANES):
                            sl = src_v[j, pl.ds(v * LANES, LANES)]
                            src_v[j, pl.ds(v * LANES, LANES)] = sl + part_off
                gds = [
                    pltpu.async_copy(y_hbm.at[src_v.at[j]], rows_v.at[j], gsem)
                    for j in range(UCH)
                ]
                for d in gds:
                    d.wait()
                sds = [
                    pltpu.async_copy(rows_v.at[j], zacc.at[dst_v.at[j]],
                                     ssem, add=True)
                    for j in range(UCH)
                ]
                for d in sds:
                    d.wait()
                return carry

            lax.fori_loop(0, n_outer, body, 0)
            plsc.subcore_barrier()
            pltpu.sync_copy(zacc.at[pl.ds(s * NPT, NPT)],
                            out_hbm.at[pl.ds(slot * NP + s * NPT, NPT)])
            plsc.subcore_barrier()

    return agg


BLK = 2000
NBLK = N // BLK


def _tc0_body(hist_ref, x_ref, w1_ref, dinv_ref, y1_ref):
    deg = hist_ref[0, :, 0:1] + hist_ref[1, :, 0:1] + 1.0
    dinv = lax.rsqrt(deg)
    dinv_ref[...] = dinv
    xw = jnp.dot(x_ref[...], w1_ref[...], preferred_element_type=jnp.float32)
    y1_ref[...] = xw * dinv


def _tc0(hist, x, w1):
    return pl.pallas_call(
        _tc0_body,
        grid=(NBLK,),
        in_specs=[
            pl.BlockSpec((2, BLK, LANES), lambda i: (0, i, 0)),
            pl.BlockSpec((BLK, 5), lambda i: (i, 0)),
            pl.BlockSpec((5, 16), lambda i: (0, 0)),
        ],
        out_specs=[
            pl.BlockSpec((BLK, 1), lambda i: (i, 0)),
            pl.BlockSpec((BLK, 16), lambda i: (i, 0)),
        ],
        out_shape=[
            jax.ShapeDtypeStruct((N, 1), jnp.float32),
            jax.ShapeDtypeStruct((N, 16), jnp.float32),
        ],
    )(hist, x, w1)


def _make_tc_mid(pin, pout, fin, fout, sum_slots):
    def body(z_ref, y_ref, dinv_ref, w_ref, b_ref, out_ref):
        if sum_slots:
            z = z_ref[0] + z_ref[1]
            y = y_ref[0]
        else:
            z = jnp.concatenate([z_ref[p] for p in range(pin)], axis=1)
            y = jnp.concatenate([y_ref[p] for p in range(pin)], axis=1)
        dinv = dinv_ref[...]
        h = jnp.maximum(dinv * (z + y) + b_ref[...], 0.0)
        yn = jnp.dot(h, w_ref[...], preferred_element_type=jnp.float32) * dinv
        for p in range(pout):
            out_ref[p] = yn[:, p * LANES:(p + 1) * LANES]

    zin = 2 if sum_slots else pin
    yin = 1 if sum_slots else pin

    def call(z, y, dinv, w, b):
        return pl.pallas_call(
            body,
            grid=(NBLK,),
            in_specs=[
                pl.BlockSpec((zin, BLK, LANES), lambda i: (0, i, 0)),
                pl.BlockSpec((yin, BLK, LANES), lambda i: (0, i, 0)),
                pl.BlockSpec((BLK, 1), lambda i: (i, 0)),
                pl.BlockSpec((fin, fout), lambda i: (0, 0)),
                pl.BlockSpec((1, fin), lambda i: (0, 0)),
            ],
            out_specs=pl.BlockSpec((pout, BLK, LANES), lambda i: (0, i, 0)),
            out_shape=jax.ShapeDtypeStruct((pout, N, LANES), jnp.float32),
        )(z.reshape(zin, NP, LANES), y.reshape(yin, N, LANES), dinv, w,
          b.reshape(1, fin))

    return call


_tc_mid12 = _make_tc_mid(1, 2, 16, 32, sum_slots=True)
_tc_mid23 = _make_tc_mid(2, 4, 32, 64, sum_slots=False)


def _pool_body(z_ref, y_ref, dinv_ref, b_ref, batch_ref, sums_ref, cnts_ref):
    i = pl.program_id(0)
    z = jnp.concatenate([z_ref[p] for p in range(4)], axis=1)
    y = jnp.concatenate([y_ref[p] for p in range(4)], axis=1)
    h = jnp.maximum(dinv_ref[...] * (z + y) + b_ref[...], 0.0)
    bt = batch_ref[...]
    io = lax.broadcasted_iota(jnp.int32, (BLK, G), 1)
    oh = (bt == io).astype(jnp.float32)
    dn = (((0,), (0,)), ((), ()))
    ps = lax.dot_general(oh, h, dn, preferred_element_type=jnp.float32)
    pc = lax.dot_general(oh, jnp.ones((BLK, 1), jnp.float32), dn,
                         preferred_element_type=jnp.float32)

    @pl.when(i == 0)
    def _():
        sums_ref[...] = ps
        cnts_ref[...] = pc

    @pl.when(i != 0)
    def _():
        sums_ref[...] += ps
        cnts_ref[...] += pc


def _tc_pool(z3, y3, dinv, b3, batch2):
    return pl.pallas_call(
        _pool_body,
        grid=(NBLK,),
        in_specs=[
            pl.BlockSpec((4, BLK, LANES), lambda i: (0, i, 0)),
            pl.BlockSpec((4, BLK, LANES), lambda i: (0, i, 0)),
            pl.BlockSpec((BLK, 1), lambda i: (i, 0)),
            pl.BlockSpec((1, 64), lambda i: (0, 0)),
            pl.BlockSpec((BLK, 1), lambda i: (i, 0)),
        ],
        out_specs=[
            pl.BlockSpec((G, 64), lambda i: (0, 0)),
            pl.BlockSpec((G, 1), lambda i: (0, 0)),
        ],
        out_shape=[
            jax.ShapeDtypeStruct((G, 64), jnp.float32),
            jax.ShapeDtypeStruct((G, 1), jnp.float32),
        ],
    )(z3.reshape(4, NP, LANES), y3.reshape(4, N, LANES), dinv,
      b3.reshape(1, 64), batch2)


def _head_body(sums_ref, cnts_ref, wfc_ref, bfc_ref, out_ref):
    pooled = sums_ref[...] / jnp.maximum(cnts_ref[...], 1.0)
    logits = jnp.dot(pooled, wfc_ref[...],
                     preferred_element_type=jnp.float32) + bfc_ref[...]
    m = jnp.max(logits, axis=1, keepdims=True)
    e = jnp.exp(logits - m)
    lse = jnp.log(jnp.sum(e, axis=1, keepdims=True)) + m
    out_ref[...] = logits - lse


def _tc_head(sums, cnts, wfc, bfc):
    return pl.pallas_call(
        _head_body,
        out_shape=jax.ShapeDtypeStruct((G, 2), jnp.float32),
    )(sums, cnts, wfc, bfc.reshape(1, 2))


def kernel(x, edge_index, batch, W1, b1, W2, b2, W3, b3, Wfc, bfc):
    src = edge_index[0]
    dst = edge_index[1]
    pad = EPAD - E
    srcp = jnp.concatenate(
        [src, jnp.zeros((pad,), jnp.int32)]).reshape(ESUB, SUB)
    dstp = jnp.concatenate(
        [dst, N + (jnp.arange(pad, dtype=jnp.int32) % NSINK)]
    ).reshape(ESUB, SUB)
    zeros16 = jnp.zeros((NP, LANES), jnp.float32)
    ones16 = jnp.ones((N, LANES), jnp.float32)

    hist = _make_agg(1, True)(ones16, srcp, dstp, zeros16).reshape(2, NP, LANES)
    dinv, y1 = _tc0(hist, x, W1)
    z1 = _make_agg(1, True)(y1, srcp, dstp, zeros16)
    y2 = _tc_mid12(z1, y1, dinv, W2, b1)
    z2 = _make_agg(2, False)(y2.reshape(2 * N, LANES), srcp, dstp, zeros16)
    y3 = _tc_mid23(z2, y2, dinv, W3, b2)
    z3 = _make_agg(4, False)(y3.reshape(4 * N, LANES), srcp, dstp, zeros16)
    sums, cnts = _tc_pool(z3, y3, dinv, b3, batch.reshape(N, 1))
    return _tc_head(sums, cnts, Wfc, bfc)


# speedup vs baseline: 15.4510x; 1.1307x over previous
import functools

import jax
import jax.numpy as jnp
from jax import lax
from jax.experimental import pallas as pl
from jax.experimental.pallas import tpu as pltpu
from jax.experimental.pallas import tpu_sc as plsc

N = 100000
E = 1600000
G = 128

NC, NS, LANES = 2, 16, 16
SUB = 128
UCH = 8
EPAD = 1638400
ESUB = EPAD // SUB
NP = 100096
NZ = NP
NPT = NP // NS
NSINK = 64

_SC_PARAMS = pltpu.CompilerParams(use_tc_tiling_on_sc=False)


@functools.cache
def _mesh():
    return plsc.VectorSubcoreMesh(
        core_axis_name="c", subcore_axis_name="s",
        num_cores=NC, num_subcores=NS)


@functools.cache
def _make_agg(nparts, edge_split):
    tasks = 1 if edge_split else nparts // NC
    nslots = 2 if edge_split else nparts
    sub_per_task = (ESUB // 2) if edge_split else ESUB
    sub_per_tile = sub_per_task // NS
    n_outer = sub_per_tile // UCH

    @functools.partial(
        pl.kernel,
        out_type=jax.ShapeDtypeStruct((nslots * NP, LANES), jnp.float32),
        mesh=_mesh(),
        compiler_params=_SC_PARAMS,
        scratch_types=[
            pltpu.VMEM((UCH, SUB), jnp.int32),
            pltpu.VMEM((UCH, SUB), jnp.int32),
            pltpu.VMEM((UCH, SUB, LANES), jnp.float32),
            pltpu.VMEM_SHARED((NZ, LANES), jnp.float32),
            pltpu.SemaphoreType.DMA,
            pltpu.SemaphoreType.DMA,
        ],
    )
    def agg(y_hbm, srcr_hbm, dstr_hbm, zeros_hbm, out_hbm,
            src_v, dst_v, rows_v, zacc, gsem, ssem):
        c = lax.axis_index("c")
        s = lax.axis_index("s")
        for q in range(tasks):
            if edge_split:
                part_off = None
                slot = c
                task_base = c * sub_per_task
            else:
                part = c * tasks + q
                part_off = part * N
                slot = part
                task_base = 0
            pltpu.sync_copy(zeros_hbm.at[pl.ds(s * NPT, NPT)],
                            zacc.at[pl.ds(s * NPT, NPT)])
            plsc.subcore_barrier()
            tile_base = task_base + s * sub_per_tile

            def body(k, carry, part_off=part_off, tile_base=tile_base):
                @pl.when(k > 0)
                def _():
                    for j in range(UCH):
                        pltpu.make_async_copy(
                            rows_v.at[j], zacc.at[dst_v.at[j]], ssem).wait()
                base = tile_base + k * UCH
                pltpu.sync_copy(srcr_hbm.at[pl.ds(base, UCH)], src_v)
                pltpu.sync_copy(dstr_hbm.at[pl.ds(base, UCH)], dst_v)
                if part_off is not None:
                    for j in range(UCH):
                        for v in range(SUB // LANES):
                            sl = src_v[j, pl.ds(v * LANES, LANES)]
                            src_v[j, pl.ds(v * LANES, LANES)] = sl + part_off
                gds = [
                    pltpu.async_copy(y_hbm.at[src_v.at[j]], rows_v.at[j], gsem)
                    for j in range(UCH)
                ]
                for j in range(UCH):
                    gds[j].wait()
                    pltpu.async_copy(rows_v.at[j], zacc.at[dst_v.at[j]],
                                     ssem, add=True)
                return carry

            lax.fori_loop(0, n_outer, body, 0)
            for j in range(UCH):
                pltpu.make_async_copy(
                    rows_v.at[j], zacc.at[dst_v.at[j]], ssem).wait()
            plsc.subcore_barrier()
            pltpu.sync_copy(zacc.at[pl.ds(s * NPT, NPT)],
                            out_hbm.at[pl.ds(slot * NP + s * NPT, NPT)])
            plsc.subcore_barrier()

    return agg


@functools.cache
def _get_deg_kernel():
    sub_per_tile = (ESUB // 2) // NS
    n_outer = sub_per_tile // UCH

    @functools.partial(
        pl.kernel,
        out_type=jax.ShapeDtypeStruct((2 * NP, LANES), jnp.float32),
        mesh=_mesh(),
        compiler_params=_SC_PARAMS,
        scratch_types=[
            pltpu.VMEM((UCH, SUB), jnp.int32),
            pltpu.VMEM((SUB, LANES), jnp.float32),
            pltpu.VMEM_SHARED((NZ, LANES), jnp.float32),
            pltpu.SemaphoreType.DMA,
        ],
    )
    def deg(ones_hbm, dstr_hbm, zeros_hbm, out_hbm, dst_v, ones_v, zacc, ssem):
        c = lax.axis_index("c")
        s = lax.axis_index("s")
        pltpu.sync_copy(ones_hbm, ones_v)
        pltpu.sync_copy(zeros_hbm.at[pl.ds(s * NPT, NPT)],
                        zacc.at[pl.ds(s * NPT, NPT)])
        plsc.subcore_barrier()
        tile_base = c * (ESUB // 2) + s * sub_per_tile

        def body(k, carry):
            @pl.when(k > 0)
            def _():
                for j in range(UCH):
                    pltpu.make_async_copy(
                        ones_v, zacc.at[dst_v.at[j]], ssem).wait()
            pltpu.sync_copy(dstr_hbm.at[pl.ds(tile_base + k * UCH, UCH)],
                            dst_v)
            for j in range(UCH):
                pltpu.async_copy(ones_v, zacc.at[dst_v.at[j]], ssem, add=True)
            return carry

        lax.fori_loop(0, n_outer, body, 0)
        for j in range(UCH):
            pltpu.make_async_copy(ones_v, zacc.at[dst_v.at[j]], ssem).wait()
        plsc.subcore_barrier()
        pltpu.sync_copy(zacc.at[pl.ds(s * NPT, NPT)],
                        out_hbm.at[pl.ds(c * NP + s * NPT, NPT)])

    return deg


BLK = 2000
NBLK = N // BLK


def _tc0_body(hist_ref, x_ref, w1_ref, dinv_ref, y1_ref):
    deg = hist_ref[0, :, 0:1] + hist_ref[1, :, 0:1] + 1.0
    dinv = lax.rsqrt(deg)
    dinv_ref[...] = dinv
    xw = jnp.dot(x_ref[...], w1_ref[...], preferred_element_type=jnp.float32)
    y1_ref[...] = xw * dinv


def _tc0(hist, x, w1):
    return pl.pallas_call(
        _tc0_body,
        grid=(NBLK,),
        in_specs=[
            pl.BlockSpec((2, BLK, LANES), lambda i: (0, i, 0)),
            pl.BlockSpec((BLK, 5), lambda i: (i, 0)),
            pl.BlockSpec((5, 16), lambda i: (0, 0)),
        ],
        out_specs=[
            pl.BlockSpec((BLK, 1), lambda i: (i, 0)),
            pl.BlockSpec((BLK, 16), lambda i: (i, 0)),
        ],
        out_shape=[
            jax.ShapeDtypeStruct((N, 1), jnp.float32),
            jax.ShapeDtypeStruct((N, 16), jnp.float32),
        ],
    )(hist, x, w1)


def _make_tc_mid(pin, pout, fin, fout, sum_slots):
    def body(z_ref, y_ref, dinv_ref, w_ref, b_ref, out_ref):
        if sum_slots:
            z = z_ref[0] + z_ref[1]
            y = y_ref[0]
        else:
            z = jnp.concatenate([z_ref[p] for p in range(pin)], axis=1)
            y = jnp.concatenate([y_ref[p] for p in range(pin)], axis=1)
        dinv = dinv_ref[...]
        h = jnp.maximum(dinv * (z + y) + b_ref[...], 0.0)
        yn = jnp.dot(h, w_ref[...], preferred_element_type=jnp.float32) * dinv
        for p in range(pout):
            out_ref[p] = yn[:, p * LANES:(p + 1) * LANES]

    zin = 2 if sum_slots else pin
    yin = 1 if sum_slots else pin

    def call(z, y, dinv, w, b):
        return pl.pallas_call(
            body,
            grid=(NBLK,),
            in_specs=[
                pl.BlockSpec((zin, BLK, LANES), lambda i: (0, i, 0)),
                pl.BlockSpec((yin, BLK, LANES), lambda i: (0, i, 0)),
                pl.BlockSpec((BLK, 1), lambda i: (i, 0)),
                pl.BlockSpec((fin, fout), lambda i: (0, 0)),
                pl.BlockSpec((1, fin), lambda i: (0, 0)),
            ],
            out_specs=pl.BlockSpec((pout, BLK, LANES), lambda i: (0, i, 0)),
            out_shape=jax.ShapeDtypeStruct((pout, N, LANES), jnp.float32),
        )(z.reshape(zin, NP, LANES), y.reshape(yin, N, LANES), dinv, w,
          b.reshape(1, fin))

    return call


_tc_mid12 = _make_tc_mid(1, 2, 16, 32, sum_slots=True)
_tc_mid23 = _make_tc_mid(2, 4, 32, 64, sum_slots=False)


def _pool_body(z_ref, y_ref, dinv_ref, b_ref, batch_ref, sums_ref, cnts_ref):
    i = pl.program_id(0)
    z = jnp.concatenate([z_ref[p] for p in range(4)], axis=1)
    y = jnp.concatenate([y_ref[p] for p in range(4)], axis=1)
    h = jnp.maximum(dinv_ref[...] * (z + y) + b_ref[...], 0.0)
    bt = batch_ref[...]
    io = lax.broadcasted_iota(jnp.int32, (BLK, G), 1)
    oh = (bt == io).astype(jnp.float32)
    dn = (((0,), (0,)), ((), ()))
    ps = lax.dot_general(oh, h, dn, preferred_element_type=jnp.float32)
    pc = lax.dot_general(oh, jnp.ones((BLK, 1), jnp.float32), dn,
                         preferred_element_type=jnp.float32)

    @pl.when(i == 0)
    def _():
        sums_ref[...] = ps
        cnts_ref[...] = pc

    @pl.when(i != 0)
    def _():
        sums_ref[...] += ps
        cnts_ref[...] += pc


def _tc_pool(z3, y3, dinv, b3, batch2):
    return pl.pallas_call(
        _pool_body,
        grid=(NBLK,),
        in_specs=[
            pl.BlockSpec((4, BLK, LANES), lambda i: (0, i, 0)),
            pl.BlockSpec((4, BLK, LANES), lambda i: (0, i, 0)),
            pl.BlockSpec((BLK, 1), lambda i: (i, 0)),
            pl.BlockSpec((1, 64), lambda i: (0, 0)),
            pl.BlockSpec((BLK, 1), lambda i: (i, 0)),
        ],
        out_specs=[
            pl.BlockSpec((G, 64), lambda i: (0, 0)),
            pl.BlockSpec((G, 1), lambda i: (0, 0)),
        ],
        out_shape=[
            jax.ShapeDtypeStruct((G, 64), jnp.float32),
            jax.ShapeDtypeStruct((G, 1), jnp.float32),
        ],
    )(z3.reshape(4, NP, LANES), y3.reshape(4, N, LANES), dinv,
      b3.reshape(1, 64), batch2)


def _head_body(sums_ref, cnts_ref, wfc_ref, bfc_ref, out_ref):
    pooled = sums_ref[...] / jnp.maximum(cnts_ref[...], 1.0)
    logits = jnp.dot(pooled, wfc_ref[...],
                     preferred_element_type=jnp.float32) + bfc_ref[...]
    m = jnp.max(logits, axis=1, keepdims=True)
    e = jnp.exp(logits - m)
    lse = jnp.log(jnp.sum(e, axis=1, keepdims=True)) + m
    out_ref[...] = logits - lse


def _tc_head(sums, cnts, wfc, bfc):
    return pl.pallas_call(
        _head_body,
        out_shape=jax.ShapeDtypeStruct((G, 2), jnp.float32),
    )(sums, cnts, wfc, bfc.reshape(1, 2))


def kernel(x, edge_index, batch, W1, b1, W2, b2, W3, b3, Wfc, bfc):
    src = edge_index[0]
    dst = edge_index[1]
    pad = EPAD - E
    srcp = jnp.concatenate(
        [src, jnp.zeros((pad,), jnp.int32)]).reshape(ESUB, SUB)
    dstp = jnp.concatenate(
        [dst, N + (jnp.arange(pad, dtype=jnp.int32) % NSINK)]
    ).reshape(ESUB, SUB)
    zeros16 = jnp.zeros((NP, LANES), jnp.float32)
    ones16 = jnp.ones((SUB, LANES), jnp.float32)

    hist = _get_deg_kernel()(ones16, dstp, zeros16).reshape(2, NP, LANES)
    dinv, y1 = _tc0(hist, x, W1)
    z1 = _make_agg(1, True)(y1, srcp, dstp, zeros16)
    y2 = _tc_mid12(z1, y1, dinv, W2, b1)
    z2 = _make_agg(2, False)(y2.reshape(2 * N, LANES), srcp, dstp, zeros16)
    y3 = _tc_mid23(z2, y2, dinv, W3, b2)
    z3 = _make_agg(4, False)(y3.reshape(4 * N, LANES), srcp, dstp, zeros16)
    sums, cnts = _tc_pool(z3, y3, dinv, b3, batch.reshape(N, 1))
    return _tc_head(sums, cnts, Wfc, bfc)


# speedup vs baseline: 15.9768x; 1.0340x over previous
import functools

import jax
import jax.numpy as jnp
from jax import lax
from jax.experimental import pallas as pl
from jax.experimental.pallas import tpu as pltpu
from jax.experimental.pallas import tpu_sc as plsc

N = 100000
E = 1600000
G = 128

NC, NS, LANES = 2, 16, 16
SUB = 128
UCH = 10
EPAD = 1638400
ESUB = EPAD // SUB
NP = 100096
NZ = NP
NPT = NP // NS
NSINK = 64

_SC_PARAMS = pltpu.CompilerParams(use_tc_tiling_on_sc=False)


@functools.cache
def _mesh():
    return plsc.VectorSubcoreMesh(
        core_axis_name="c", subcore_axis_name="s",
        num_cores=NC, num_subcores=NS)


@functools.cache
def _make_agg(nparts, edge_split):
    tasks = 1 if edge_split else nparts // NC
    nslots = 2 if edge_split else nparts
    sub_per_task = (ESUB // 2) if edge_split else ESUB
    sub_per_tile = sub_per_task // NS
    n_outer = sub_per_tile // UCH

    @functools.partial(
        pl.kernel,
        out_type=jax.ShapeDtypeStruct((nslots * NP, LANES), jnp.float32),
        mesh=_mesh(),
        compiler_params=_SC_PARAMS,
        scratch_types=[
            pltpu.VMEM((UCH, SUB), jnp.int32),
            pltpu.VMEM((UCH, SUB), jnp.int32),
            pltpu.VMEM((UCH, SUB, LANES), jnp.float32),
            pltpu.VMEM_SHARED((NZ, LANES), jnp.float32),
            pltpu.SemaphoreType.DMA,
            pltpu.SemaphoreType.DMA,
        ],
    )
    def agg(y_hbm, srcr_hbm, dstr_hbm, zeros_hbm, out_hbm,
            src_v, dst_v, rows_v, zacc, gsem, ssem):
        c = lax.axis_index("c")
        s = lax.axis_index("s")
        for q in range(tasks):
            if edge_split:
                part_off = None
                slot = c
                task_base = c * sub_per_task
            else:
                part = c * tasks + q
                part_off = part * N
                slot = part
                task_base = 0
            pltpu.sync_copy(zeros_hbm.at[pl.ds(s * NPT, NPT)],
                            zacc.at[pl.ds(s * NPT, NPT)])
            plsc.subcore_barrier()
            tile_base = task_base + s * sub_per_tile

            def body(k, carry, part_off=part_off, tile_base=tile_base):
                @pl.when(k > 0)
                def _():
                    for j in range(UCH):
                        pltpu.make_async_copy(
                            rows_v.at[j], zacc.at[dst_v.at[j]], ssem).wait()
                base = tile_base + k * UCH
                pltpu.sync_copy(srcr_hbm.at[pl.ds(base, UCH)], src_v)
                pltpu.sync_copy(dstr_hbm.at[pl.ds(base, UCH)], dst_v)
                if part_off is not None:
                    for j in range(UCH):
                        for v in range(SUB // LANES):
                            sl = src_v[j, pl.ds(v * LANES, LANES)]
                            src_v[j, pl.ds(v * LANES, LANES)] = sl + part_off
                gds = [
                    pltpu.async_copy(y_hbm.at[src_v.at[j]], rows_v.at[j], gsem)
                    for j in range(UCH)
                ]
                for j in range(UCH):
                    gds[j].wait()
                    pltpu.async_copy(rows_v.at[j], zacc.at[dst_v.at[j]],
                                     ssem, add=True)
                return carry

            lax.fori_loop(0, n_outer, body, 0)
            for j in range(UCH):
                pltpu.make_async_copy(
                    rows_v.at[j], zacc.at[dst_v.at[j]], ssem).wait()
            plsc.subcore_barrier()
            pltpu.sync_copy(zacc.at[pl.ds(s * NPT, NPT)],
                            out_hbm.at[pl.ds(slot * NP + s * NPT, NPT)])
            plsc.subcore_barrier()

    return agg


@functools.cache
def _get_deg_kernel():
    sub_per_tile = (ESUB // 2) // NS
    n_outer = sub_per_tile // UCH

    @functools.partial(
        pl.kernel,
        out_type=jax.ShapeDtypeStruct((2 * NP, LANES), jnp.float32),
        mesh=_mesh(),
        compiler_params=_SC_PARAMS,
        scratch_types=[
            pltpu.VMEM((UCH, SUB), jnp.int32),
            pltpu.VMEM((SUB, LANES), jnp.float32),
            pltpu.VMEM_SHARED((NZ, LANES), jnp.float32),
            pltpu.SemaphoreType.DMA,
        ],
    )
    def deg(ones_hbm, dstr_hbm, zeros_hbm, out_hbm, dst_v, ones_v, zacc, ssem):
        c = lax.axis_index("c")
        s = lax.axis_index("s")
        pltpu.sync_copy(ones_hbm, ones_v)
        pltpu.sync_copy(zeros_hbm.at[pl.ds(s * NPT, NPT)],
                        zacc.at[pl.ds(s * NPT, NPT)])
        plsc.subcore_barrier()
        tile_base = c * (ESUB // 2) + s * sub_per_tile

        def body(k, carry):
            @pl.when(k > 0)
            def _():
                for j in range(UCH):
                    pltpu.make_async_copy(
                        ones_v, zacc.at[dst_v.at[j]], ssem).wait()
            pltpu.sync_copy(dstr_hbm.at[pl.ds(tile_base + k * UCH, UCH)],
                            dst_v)
            for j in range(UCH):
                pltpu.async_copy(ones_v, zacc.at[dst_v.at[j]], ssem, add=True)
            return carry

        lax.fori_loop(0, n_outer, body, 0)
        for j in range(UCH):
            pltpu.make_async_copy(ones_v, zacc.at[dst_v.at[j]], ssem).wait()
        plsc.subcore_barrier()
        pltpu.sync_copy(zacc.at[pl.ds(s * NPT, NPT)],
                        out_hbm.at[pl.ds(c * NP + s * NPT, NPT)])

    return deg


BLK = 2000
NBLK = N // BLK


def _tc0_body(hist_ref, x_ref, w1_ref, dinv_ref, y1_ref):
    deg = hist_ref[0, :, 0:1] + hist_ref[1, :, 0:1] + 1.0
    dinv = lax.rsqrt(deg)
    dinv_ref[...] = dinv
    xw = jnp.dot(x_ref[...], w1_ref[...], preferred_element_type=jnp.float32)
    y1_ref[...] = xw * dinv


def _tc0(hist, x, w1):
    return pl.pallas_call(
        _tc0_body,
        grid=(NBLK,),
        in_specs=[
            pl.BlockSpec((2, BLK, LANES), lambda i: (0, i, 0)),
            pl.BlockSpec((BLK, 5), lambda i: (i, 0)),
            pl.BlockSpec((5, 16), lambda i: (0, 0)),
        ],
        out_specs=[
            pl.BlockSpec((BLK, 1), lambda i: (i, 0)),
            pl.BlockSpec((BLK, 16), lambda i: (i, 0)),
        ],
        out_shape=[
            jax.ShapeDtypeStruct((N, 1), jnp.float32),
            jax.ShapeDtypeStruct((N, 16), jnp.float32),
        ],
    )(hist, x, w1)


def _make_tc_mid(pin, pout, fin, fout, sum_slots):
    def body(z_ref, y_ref, dinv_ref, w_ref, b_ref, out_ref):
        if sum_slots:
            z = z_ref[0] + z_ref[1]
            y = y_ref[0]
        else:
            z = jnp.concatenate([z_ref[p] for p in range(pin)], axis=1)
            y = jnp.concatenate([y_ref[p] for p in range(pin)], axis=1)
        dinv = dinv_ref[...]
        h = jnp.maximum(dinv * (z + y) + b_ref[...], 0.0)
        yn = jnp.dot(h, w_ref[...], preferred_element_type=jnp.float32) * dinv
        for p in range(pout):
            out_ref[p] = yn[:, p * LANES:(p + 1) * LANES]

    zin = 2 if sum_slots else pin
    yin = 1 if sum_slots else pin

    def call(z, y, dinv, w, b):
        return pl.pallas_call(
            body,
            grid=(NBLK,),
            in_specs=[
                pl.BlockSpec((zin, BLK, LANES), lambda i: (0, i, 0)),
                pl.BlockSpec((yin, BLK, LANES), lambda i: (0, i, 0)),
                pl.BlockSpec((BLK, 1), lambda i: (i, 0)),
                pl.BlockSpec((fin, fout), lambda i: (0, 0)),
                pl.BlockSpec((1, fin), lambda i: (0, 0)),
            ],
            out_specs=pl.BlockSpec((pout, BLK, LANES), lambda i: (0, i, 0)),
            out_shape=jax.ShapeDtypeStruct((pout, N, LANES), jnp.float32),
        )(z.reshape(zin, NP, LANES), y.reshape(yin, N, LANES), dinv, w,
          b.reshape(1, fin))

    return call


_tc_mid12 = _make_tc_mid(1, 2, 16, 32, sum_slots=True)
_tc_mid23 = _make_tc_mid(2, 4, 32, 64, sum_slots=False)


def _pool_body(z_ref, y_ref, dinv_ref, b_ref, batch_ref, sums_ref, cnts_ref):
    i = pl.program_id(0)
    z = jnp.concatenate([z_ref[p] for p in range(4)], axis=1)
    y = jnp.concatenate([y_ref[p] for p in range(4)], axis=1)
    h = jnp.maximum(dinv_ref[...] * (z + y) + b_ref[...], 0.0)
    bt = batch_ref[...]
    io = lax.broadcasted_iota(jnp.int32, (BLK, G), 1)
    oh = (bt == io).astype(jnp.float32)
    dn = (((0,), (0,)), ((), ()))
    ps = lax.dot_general(oh, h, dn, preferred_element_type=jnp.float32)
    pc = lax.dot_general(oh, jnp.ones((BLK, 1), jnp.float32), dn,
                         preferred_element_type=jnp.float32)

    @pl.when(i == 0)
    def _():
        sums_ref[...] = ps
        cnts_ref[...] = pc

    @pl.when(i != 0)
    def _():
        sums_ref[...] += ps
        cnts_ref[...] += pc


def _tc_pool(z3, y3, dinv, b3, batch2):
    return pl.pallas_call(
        _pool_body,
        grid=(NBLK,),
        in_specs=[
            pl.BlockSpec((4, BLK, LANES), lambda i: (0, i, 0)),
            pl.BlockSpec((4, BLK, LANES), lambda i: (0, i, 0)),
            pl.BlockSpec((BLK, 1), lambda i: (i, 0)),
            pl.BlockSpec((1, 64), lambda i: (0, 0)),
            pl.BlockSpec((BLK, 1), lambda i: (i, 0)),
        ],
        out_specs=[
            pl.BlockSpec((G, 64), lambda i: (0, 0)),
            pl.BlockSpec((G, 1), lambda i: (0, 0)),
        ],
        out_shape=[
            jax.ShapeDtypeStruct((G, 64), jnp.float32),
            jax.ShapeDtypeStruct((G, 1), jnp.float32),
        ],
    )(z3.reshape(4, NP, LANES), y3.reshape(4, N, LANES), dinv,
      b3.reshape(1, 64), batch2)


def _head_body(sums_ref, cnts_ref, wfc_ref, bfc_ref, out_ref):
    pooled = sums_ref[...] / jnp.maximum(cnts_ref[...], 1.0)
    logits = jnp.dot(pooled, wfc_ref[...],
                     preferred_element_type=jnp.float32) + bfc_ref[...]
    m = jnp.max(logits, axis=1, keepdims=True)
    e = jnp.exp(logits - m)
    lse = jnp.log(jnp.sum(e, axis=1, keepdims=True)) + m
    out_ref[...] = logits - lse


def _tc_head(sums, cnts, wfc, bfc):
    return pl.pallas_call(
        _head_body,
        out_shape=jax.ShapeDtypeStruct((G, 2), jnp.float32),
    )(sums, cnts, wfc, bfc.reshape(1, 2))


def kernel(x, edge_index, batch, W1, b1, W2, b2, W3, b3, Wfc, bfc):
    src = edge_index[0]
    dst = edge_index[1]
    pad = EPAD - E
    srcp = jnp.concatenate(
        [src, jnp.zeros((pad,), jnp.int32)]).reshape(ESUB, SUB)
    dstp = jnp.concatenate(
        [dst, N + (jnp.arange(pad, dtype=jnp.int32) % NSINK)]
    ).reshape(ESUB, SUB)
    zeros16 = jnp.zeros((NP, LANES), jnp.float32)
    ones16 = jnp.ones((SUB, LANES), jnp.float32)

    hist = _get_deg_kernel()(ones16, dstp, zeros16).reshape(2, NP, LANES)
    dinv, y1 = _tc0(hist, x, W1)
    z1 = _make_agg(1, True)(y1, srcp, dstp, zeros16)
    y2 = _tc_mid12(z1, y1, dinv, W2, b1)
    z2 = _make_agg(2, False)(y2.reshape(2 * N, LANES), srcp, dstp, zeros16)
    y3 = _tc_mid23(z2, y2, dinv, W3, b2)
    z3 = _make_agg(4, False)(y3.reshape(4 * N, LANES), srcp, dstp, zeros16)
    sums, cnts = _tc_pool(z3, y3, dinv, b3, batch.reshape(N, 1))
    return _tc_head(sums, cnts, Wfc, bfc)


# speedup vs baseline: 22.7103x; 1.4215x over previous
import functools

import jax
import jax.numpy as jnp
from jax import lax
from jax.experimental import pallas as pl
from jax.experimental.pallas import tpu as pltpu
from jax.experimental.pallas import tpu_sc as plsc

N = 100000
E = 1600000
G = 128

NC, NS, LANES = 2, 16, 16
SUB = 128
UCH = 10
EPAD = 1638400
ESUB = EPAD // SUB
NP = 100096
NZ = NP
NPT = NP // NS
NSINK = 64

_SC_PARAMS = pltpu.CompilerParams(use_tc_tiling_on_sc=False)


@functools.cache
def _mesh():
    return plsc.VectorSubcoreMesh(
        core_axis_name="c", subcore_axis_name="s",
        num_cores=NC, num_subcores=NS)


@functools.cache
def _make_agg(nparts, edge_split, ncols=LANES, dtype=jnp.float32, uch=UCH):
    tasks = 1 if edge_split else nparts // NC
    nslots = 2 if edge_split else nparts
    sub_per_task = (ESUB // 2) if edge_split else ESUB
    sub_per_tile = sub_per_task // NS
    n_outer = sub_per_tile // uch
    UCH = uch

    @functools.partial(
        pl.kernel,
        out_type=jax.ShapeDtypeStruct((nslots * NP, ncols), dtype),
        mesh=_mesh(),
        compiler_params=_SC_PARAMS,
        scratch_types=[
            pltpu.VMEM((UCH, SUB), jnp.int32),
            pltpu.VMEM((UCH, SUB), jnp.int32),
            pltpu.VMEM((UCH, SUB, ncols), dtype),
            pltpu.VMEM_SHARED((NZ, ncols), dtype),
            pltpu.SemaphoreType.DMA,
            pltpu.SemaphoreType.DMA,
        ],
    )
    def agg(y_hbm, srcr_hbm, dstr_hbm, zeros_hbm, out_hbm,
            src_v, dst_v, rows_v, zacc, gsem, ssem):
        c = lax.axis_index("c")
        s = lax.axis_index("s")
        for q in range(tasks):
            if edge_split:
                part_off = None
                slot = c
                task_base = c * sub_per_task
            else:
                part = c * tasks + q
                part_off = part * N
                slot = part
                task_base = 0
            pltpu.sync_copy(zeros_hbm.at[pl.ds(s * NPT, NPT)],
                            zacc.at[pl.ds(s * NPT, NPT)])
            plsc.subcore_barrier()
            tile_base = task_base + s * sub_per_tile

            def body(k, carry, part_off=part_off, tile_base=tile_base):
                @pl.when(k > 0)
                def _():
                    for j in range(UCH):
                        pltpu.make_async_copy(
                            rows_v.at[j], zacc.at[dst_v.at[j]], ssem).wait()
                base = tile_base + k * UCH
                pltpu.sync_copy(srcr_hbm.at[pl.ds(base, UCH)], src_v)
                pltpu.sync_copy(dstr_hbm.at[pl.ds(base, UCH)], dst_v)
                if part_off is not None:
                    for j in range(UCH):
                        for v in range(SUB // LANES):
                            sl = src_v[j, pl.ds(v * LANES, LANES)]
                            src_v[j, pl.ds(v * LANES, LANES)] = sl + part_off
                gds = [
                    pltpu.async_copy(y_hbm.at[src_v.at[j]], rows_v.at[j], gsem)
                    for j in range(UCH)
                ]
                for j in range(UCH):
                    gds[j].wait()
                    pltpu.async_copy(rows_v.at[j], zacc.at[dst_v.at[j]],
                                     ssem, add=True)
                return carry

            lax.fori_loop(0, n_outer, body, 0)
            for j in range(UCH):
                pltpu.make_async_copy(
                    rows_v.at[j], zacc.at[dst_v.at[j]], ssem).wait()
            plsc.subcore_barrier()
            pltpu.sync_copy(zacc.at[pl.ds(s * NPT, NPT)],
                            out_hbm.at[pl.ds(slot * NP + s * NPT, NPT)])
            plsc.subcore_barrier()

    return agg


@functools.cache
def _get_deg_kernel():
    sub_per_tile = (ESUB // 2) // NS
    n_outer = sub_per_tile // UCH

    @functools.partial(
        pl.kernel,
        out_type=jax.ShapeDtypeStruct((2 * NP, LANES), jnp.float32),
        mesh=_mesh(),
        compiler_params=_SC_PARAMS,
        scratch_types=[
            pltpu.VMEM((UCH, SUB), jnp.int32),
            pltpu.VMEM((SUB, LANES), jnp.float32),
            pltpu.VMEM_SHARED((NZ, LANES), jnp.float32),
            pltpu.SemaphoreType.DMA,
        ],
    )
    def deg(ones_hbm, dstr_hbm, zeros_hbm, out_hbm, dst_v, ones_v, zacc, ssem):
        c = lax.axis_index("c")
        s = lax.axis_index("s")
        pltpu.sync_copy(ones_hbm, ones_v)
        pltpu.sync_copy(zeros_hbm.at[pl.ds(s * NPT, NPT)],
                        zacc.at[pl.ds(s * NPT, NPT)])
        plsc.subcore_barrier()
        tile_base = c * (ESUB // 2) + s * sub_per_tile

        def body(k, carry):
            @pl.when(k > 0)
            def _():
                for j in range(UCH):
                    pltpu.make_async_copy(
                        ones_v, zacc.at[dst_v.at[j]], ssem).wait()
            pltpu.sync_copy(dstr_hbm.at[pl.ds(tile_base + k * UCH, UCH)],
                            dst_v)
            for j in range(UCH):
                pltpu.async_copy(ones_v, zacc.at[dst_v.at[j]], ssem, add=True)
            return carry

        lax.fori_loop(0, n_outer, body, 0)
        for j in range(UCH):
            pltpu.make_async_copy(ones_v, zacc.at[dst_v.at[j]], ssem).wait()
        plsc.subcore_barrier()
        pltpu.sync_copy(zacc.at[pl.ds(s * NPT, NPT)],
                        out_hbm.at[pl.ds(c * NP + s * NPT, NPT)])

    return deg


BLK = 2000
NBLK = N // BLK


def _tc0_body(hist_ref, x_ref, w1_ref, dinv_ref, y1_ref):
    deg = hist_ref[0, :, 0:1] + hist_ref[1, :, 0:1] + 1.0
    dinv = lax.rsqrt(deg)
    dinv_ref[...] = dinv
    xw = jnp.dot(x_ref[...], w1_ref[...], preferred_element_type=jnp.float32)
    y1_ref[...] = xw * dinv


def _tc0(hist, x, w1):
    return pl.pallas_call(
        _tc0_body,
        grid=(NBLK,),
        in_specs=[
            pl.BlockSpec((2, BLK, LANES), lambda i: (0, i, 0)),
            pl.BlockSpec((BLK, 5), lambda i: (i, 0)),
            pl.BlockSpec((5, 16), lambda i: (0, 0)),
        ],
        out_specs=[
            pl.BlockSpec((BLK, 1), lambda i: (i, 0)),
            pl.BlockSpec((BLK, 16), lambda i: (i, 0)),
        ],
        out_shape=[
            jax.ShapeDtypeStruct((N, 1), jnp.float32),
            jax.ShapeDtypeStruct((N, 16), jnp.float32),
        ],
    )(hist, x, w1)


def _mid12_body(z_ref, y_ref, dinv_ref, w_ref, b_ref, out_ref):
    z = z_ref[0] + z_ref[1]
    dinv = dinv_ref[...]
    h = jnp.maximum(dinv * (z + y_ref[...]) + b_ref[...], 0.0)
    yn = jnp.dot(h, w_ref[...], preferred_element_type=jnp.float32) * dinv
    out_ref[...] = yn.astype(jnp.bfloat16)


def _tc_mid12(z1, y1, dinv, w2, b1):
    return pl.pallas_call(
        _mid12_body,
        grid=(NBLK,),
        in_specs=[
            pl.BlockSpec((2, BLK, LANES), lambda i: (0, i, 0)),
            pl.BlockSpec((BLK, LANES), lambda i: (i, 0)),
            pl.BlockSpec((BLK, 1), lambda i: (i, 0)),
            pl.BlockSpec((16, 32), lambda i: (0, 0)),
            pl.BlockSpec((1, 16), lambda i: (0, 0)),
        ],
        out_specs=pl.BlockSpec((BLK, 32), lambda i: (i, 0)),
        out_shape=jax.ShapeDtypeStruct((N, 32), jnp.bfloat16),
    )(z1.reshape(2, NP, LANES), y1, dinv, w2, b1.reshape(1, 16))


def _mid23_body(z_ref, y_ref, dinv_ref, w_ref, b_ref, out_ref):
    z = (z_ref[0] + z_ref[1]).astype(jnp.float32)
    y = y_ref[...].astype(jnp.float32)
    dinv = dinv_ref[...]
    h = jnp.maximum(dinv * (z + y) + b_ref[...], 0.0)
    yn = jnp.dot(h, w_ref[...], preferred_element_type=jnp.float32) * dinv
    for p in range(2):
        out_ref[p] = yn[:, p * 32:(p + 1) * 32].astype(jnp.bfloat16)


def _tc_mid23(z2, y2, dinv, w3, b2):
    return pl.pallas_call(
        _mid23_body,
        grid=(NBLK,),
        in_specs=[
            pl.BlockSpec((2, BLK, 32), lambda i: (0, i, 0)),
            pl.BlockSpec((BLK, 32), lambda i: (i, 0)),
            pl.BlockSpec((BLK, 1), lambda i: (i, 0)),
            pl.BlockSpec((32, 64), lambda i: (0, 0)),
            pl.BlockSpec((1, 32), lambda i: (0, 0)),
        ],
        out_specs=pl.BlockSpec((2, BLK, 32), lambda i: (0, i, 0)),
        out_shape=jax.ShapeDtypeStruct((2, N, 32), jnp.bfloat16),
    )(z2.reshape(2, NP, 32), y2, dinv, w3, b2.reshape(1, 32))


def _pool_body(z_ref, y_ref, dinv_ref, b_ref, batch_ref, sums_ref, cnts_ref):
    i = pl.program_id(0)
    z = jnp.concatenate([z_ref[0], z_ref[1]], axis=1).astype(jnp.float32)
    y = jnp.concatenate([y_ref[0], y_ref[1]], axis=1).astype(jnp.float32)
    h = jnp.maximum(dinv_ref[...] * (z + y) + b_ref[...], 0.0)
    bt = batch_ref[...]
    io = lax.broadcasted_iota(jnp.int32, (BLK, G), 1)
    oh = (bt == io).astype(jnp.float32)
    dn = (((0,), (0,)), ((), ()))
    ps = lax.dot_general(oh, h, dn, preferred_element_type=jnp.float32)
    pc = lax.dot_general(oh, jnp.ones((BLK, 1), jnp.float32), dn,
                         preferred_element_type=jnp.float32)

    @pl.when(i == 0)
    def _():
        sums_ref[...] = ps
        cnts_ref[...] = pc

    @pl.when(i != 0)
    def _():
        sums_ref[...] += ps
        cnts_ref[...] += pc


def _tc_pool(z3, y3, dinv, b3, batch2):
    return pl.pallas_call(
        _pool_body,
        grid=(NBLK,),
        in_specs=[
            pl.BlockSpec((2, BLK, 32), lambda i: (0, i, 0)),
            pl.BlockSpec((2, BLK, 32), lambda i: (0, i, 0)),
            pl.BlockSpec((BLK, 1), lambda i: (i, 0)),
            pl.BlockSpec((1, 64), lambda i: (0, 0)),
            pl.BlockSpec((BLK, 1), lambda i: (i, 0)),
        ],
        out_specs=[
            pl.BlockSpec((G, 64), lambda i: (0, 0)),
            pl.BlockSpec((G, 1), lambda i: (0, 0)),
        ],
        out_shape=[
            jax.ShapeDtypeStruct((G, 64), jnp.float32),
            jax.ShapeDtypeStruct((G, 1), jnp.float32),
        ],
    )(z3.reshape(2, NP, 32), y3, dinv, b3.reshape(1, 64), batch2)


def _head_body(sums_ref, cnts_ref, wfc_ref, bfc_ref, out_ref):
    pooled = sums_ref[...] / jnp.maximum(cnts_ref[...], 1.0)
    logits = jnp.dot(pooled, wfc_ref[...],
                     preferred_element_type=jnp.float32) + bfc_ref[...]
    m = jnp.max(logits, axis=1, keepdims=True)
    e = jnp.exp(logits - m)
    lse = jnp.log(jnp.sum(e, axis=1, keepdims=True)) + m
    out_ref[...] = logits - lse


def _tc_head(sums, cnts, wfc, bfc):
    return pl.pallas_call(
        _head_body,
        out_shape=jax.ShapeDtypeStruct((G, 2), jnp.float32),
    )(sums, cnts, wfc, bfc.reshape(1, 2))


def kernel(x, edge_index, batch, W1, b1, W2, b2, W3, b3, Wfc, bfc):
    src = edge_index[0]
    dst = edge_index[1]
    pad = EPAD - E
    srcp = jnp.concatenate(
        [src, jnp.zeros((pad,), jnp.int32)]).reshape(ESUB, SUB)
    dstp = jnp.concatenate(
        [dst, N + (jnp.arange(pad, dtype=jnp.int32) % NSINK)]
    ).reshape(ESUB, SUB)
    zeros16 = jnp.zeros((NP, LANES), jnp.float32)
    zeros32b = jnp.zeros((NP, 32), jnp.bfloat16)
    ones16 = jnp.ones((SUB, LANES), jnp.float32)

    hist = _get_deg_kernel()(ones16, dstp, zeros16).reshape(2, NP, LANES)
    dinv, y1 = _tc0(hist, x, W1)
    z1 = _make_agg(1, True)(y1, srcp, dstp, zeros16)
    y2 = _tc_mid12(z1, y1, dinv, W2, b1)
    z2 = _make_agg(1, True, ncols=32, dtype=jnp.bfloat16, uch=8)(
        y2, srcp, dstp, zeros32b)
    y3 = _tc_mid23(z2, y2, dinv, W3, b2)
    z3 = _make_agg(2, False, ncols=32, dtype=jnp.bfloat16, uch=8)(
        y3.reshape(2 * N, 32), srcp, dstp, zeros32b)
    sums, cnts = _tc_pool(z3, y3, dinv, b3, batch.reshape(N, 1))
    return _tc_head(sums, cnts, Wfc, bfc)


# speedup vs baseline: 26.0488x; 1.1470x over previous
import functools

import jax
import jax.numpy as jnp
from jax import lax
from jax.experimental import pallas as pl
from jax.experimental.pallas import tpu as pltpu
from jax.experimental.pallas import tpu_sc as plsc

N = 100000
E = 1600000
G = 128

NC, NS, LANES = 2, 16, 16
SUB = 128
UCH = 10
EPAD = 1638400
ESUB = EPAD // SUB
NP = 100096
NZ = NP
NPT = NP // NS
NSINK = 64

_SC_PARAMS = pltpu.CompilerParams(use_tc_tiling_on_sc=False)


@functools.cache
def _mesh():
    return plsc.VectorSubcoreMesh(
        core_axis_name="c", subcore_axis_name="s",
        num_cores=NC, num_subcores=NS)


@functools.cache
def _make_agg(nparts, edge_split, ncols=LANES, dtype=jnp.float32, uch=UCH):
    tasks = 1 if edge_split else nparts // NC
    nslots = 2 if edge_split else nparts
    sub_per_task = (ESUB // 2) if edge_split else ESUB
    sub_per_tile = sub_per_task // NS
    n_outer = sub_per_tile // uch
    UCH = uch

    @functools.partial(
        pl.kernel,
        out_type=jax.ShapeDtypeStruct((nslots * NP, ncols), dtype),
        mesh=_mesh(),
        compiler_params=_SC_PARAMS,
        scratch_types=[
            pltpu.VMEM((UCH, SUB), jnp.int32),
            pltpu.VMEM((UCH, SUB), jnp.int32),
            pltpu.VMEM((UCH, SUB, ncols), dtype),
            pltpu.VMEM_SHARED((NZ, ncols), dtype),
            pltpu.SemaphoreType.DMA,
            pltpu.SemaphoreType.DMA,
        ],
    )
    def agg(y_hbm, srcr_hbm, dstr_hbm, zeros_hbm, out_hbm,
            src_v, dst_v, rows_v, zacc, gsem, ssem):
        c = lax.axis_index("c")
        s = lax.axis_index("s")
        for q in range(tasks):
            if edge_split:
                part_off = None
                slot = c
                task_base = c * sub_per_task
            else:
                part = c * tasks + q
                part_off = part * NP
                slot = part
                task_base = 0
            pltpu.sync_copy(zeros_hbm.at[pl.ds(s * NPT, NPT)],
                            zacc.at[pl.ds(s * NPT, NPT)])
            plsc.subcore_barrier()
            tile_base = task_base + s * sub_per_tile

            def body(k, carry, part_off=part_off, tile_base=tile_base):
                @pl.when(k > 0)
                def _():
                    for j in range(UCH):
                        pltpu.make_async_copy(
                            rows_v.at[j], zacc.at[dst_v.at[j]], ssem).wait()
                base = tile_base + k * UCH
                pltpu.sync_copy(srcr_hbm.at[pl.ds(base, UCH)], src_v)
                pltpu.sync_copy(dstr_hbm.at[pl.ds(base, UCH)], dst_v)
                if part_off is not None:
                    for j in range(UCH):
                        for v in range(SUB // LANES):
                            sl = src_v[j, pl.ds(v * LANES, LANES)]
                            src_v[j, pl.ds(v * LANES, LANES)] = sl + part_off
                gds = [
                    pltpu.async_copy(y_hbm.at[src_v.at[j]], rows_v.at[j], gsem)
                    for j in range(UCH)
                ]
                for j in range(UCH):
                    gds[j].wait()
                    pltpu.async_copy(rows_v.at[j], zacc.at[dst_v.at[j]],
                                     ssem, add=True)
                return carry

            lax.fori_loop(0, n_outer, body, 0)
            for j in range(UCH):
                pltpu.make_async_copy(
                    rows_v.at[j], zacc.at[dst_v.at[j]], ssem).wait()
            plsc.subcore_barrier()
            pltpu.sync_copy(zacc.at[pl.ds(s * NPT, NPT)],
                            out_hbm.at[pl.ds(slot * NP + s * NPT, NPT)])
            plsc.subcore_barrier()

    return agg


@functools.cache
def _get_deg_kernel():
    sub_per_tile = (ESUB // 2) // NS
    n_outer = sub_per_tile // UCH

    @functools.partial(
        pl.kernel,
        out_type=jax.ShapeDtypeStruct((2 * NP, LANES), jnp.float32),
        mesh=_mesh(),
        compiler_params=_SC_PARAMS,
        scratch_types=[
            pltpu.VMEM((UCH, SUB), jnp.int32),
            pltpu.VMEM((SUB, LANES), jnp.float32),
            pltpu.VMEM_SHARED((NZ, LANES), jnp.float32),
            pltpu.SemaphoreType.DMA,
        ],
    )
    def deg(ones_hbm, dstr_hbm, zeros_hbm, out_hbm, dst_v, ones_v, zacc, ssem):
        c = lax.axis_index("c")
        s = lax.axis_index("s")
        pltpu.sync_copy(ones_hbm, ones_v)
        pltpu.sync_copy(zeros_hbm.at[pl.ds(s * NPT, NPT)],
                        zacc.at[pl.ds(s * NPT, NPT)])
        plsc.subcore_barrier()
        tile_base = c * (ESUB // 2) + s * sub_per_tile

        def body(k, carry):
            @pl.when(k > 0)
            def _():
                for j in range(UCH):
                    pltpu.make_async_copy(
                        ones_v, zacc.at[dst_v.at[j]], ssem).wait()
            pltpu.sync_copy(dstr_hbm.at[pl.ds(tile_base + k * UCH, UCH)],
                            dst_v)
            for j in range(UCH):
                pltpu.async_copy(ones_v, zacc.at[dst_v.at[j]], ssem, add=True)
            return carry

        lax.fori_loop(0, n_outer, body, 0)
        for j in range(UCH):
            pltpu.make_async_copy(ones_v, zacc.at[dst_v.at[j]], ssem).wait()
        plsc.subcore_barrier()
        pltpu.sync_copy(zacc.at[pl.ds(s * NPT, NPT)],
                        out_hbm.at[pl.ds(c * NP + s * NPT, NPT)])

    return deg


BLK = 4352
NBLK = NP // BLK
PBLK = BLK // 8
NPF = NP // 8


def _tc0_body(hist_ref, xq_ref, w1b_ref, dinv_ref, y1_ref):
    dinv128 = lax.rsqrt(hist_ref[0] + hist_ref[1] + 1.0)
    dinv_ref[...] = dinv128
    xw = jnp.dot(xq_ref[...], w1b_ref[...],
                 preferred_element_type=jnp.float32)
    y1_ref[...] = xw * dinv128


def _tc0(hist, xq, w1big):
    return pl.pallas_call(
        _tc0_body,
        grid=(NBLK,),
        in_specs=[
            pl.BlockSpec((2, PBLK, 128), lambda i: (0, i, 0)),
            pl.BlockSpec((PBLK, 40), lambda i: (i, 0)),
            pl.BlockSpec((40, 128), lambda i: (0, 0)),
        ],
        out_specs=[
            pl.BlockSpec((PBLK, 128), lambda i: (i, 0)),
            pl.BlockSpec((PBLK, 128), lambda i: (i, 0)),
        ],
        out_shape=[
            jax.ShapeDtypeStruct((NPF, 128), jnp.float32),
            jax.ShapeDtypeStruct((NPF, 128), jnp.float32),
        ],
    )(hist, xq, w1big)


def _mid12_body(z_ref, y_ref, dinv_ref, w_ref, b_ref, s_ref, out_ref):
    dinv128 = dinv_ref[...]
    h = jnp.maximum(dinv128 * (z_ref[0] + z_ref[1] + y_ref[...])
                    + b_ref[...], 0.0)
    dv256 = jnp.dot(dinv128, s_ref[...],
                    preferred_element_type=jnp.float32)
    yn = jnp.dot(h, w_ref[...], preferred_element_type=jnp.float32) * dv256
    out_ref[...] = yn.astype(jnp.bfloat16)


def _tc_mid12(z1p, y1p, dinv128, w2big, b1t, s256):
    return pl.pallas_call(
        _mid12_body,
        grid=(NBLK,),
        in_specs=[
            pl.BlockSpec((2, PBLK, 128), lambda i: (0, i, 0)),
            pl.BlockSpec((PBLK, 128), lambda i: (i, 0)),
            pl.BlockSpec((PBLK, 128), lambda i: (i, 0)),
            pl.BlockSpec((128, 256), lambda i: (0, 0)),
            pl.BlockSpec((1, 128), lambda i: (0, 0)),
            pl.BlockSpec((128, 256), lambda i: (0, 0)),
        ],
        out_specs=pl.BlockSpec((PBLK, 256), lambda i: (i, 0)),
        out_shape=jax.ShapeDtypeStruct((NPF, 256), jnp.bfloat16),
    )(z1p, y1p, dinv128, w2big, b1t, s256)


def _mid23_body(z_ref, y_ref, dinv_ref, w_ref, b_ref, out_ref):
    z = (z_ref[0] + z_ref[1]).astype(jnp.float32)
    y = y_ref[...].astype(jnp.float32)
    dv = dinv_ref[...]
    h = jnp.maximum(dv * (z + y) + b_ref[...], 0.0)
    yn = jnp.dot(h, w_ref[...], preferred_element_type=jnp.float32) * dv
    for p in range(2):
        out_ref[p] = yn[:, p * 32:(p + 1) * 32].astype(jnp.bfloat16)


def _tc_mid23(z2, y2, dinv_nm, w3, b2):
    return pl.pallas_call(
        _mid23_body,
        grid=(NBLK,),
        in_specs=[
            pl.BlockSpec((2, BLK, 32), lambda i: (0, i, 0)),
            pl.BlockSpec((BLK, 32), lambda i: (i, 0)),
            pl.BlockSpec((BLK, 1), lambda i: (i, 0)),
            pl.BlockSpec((32, 64), lambda i: (0, 0)),
            pl.BlockSpec((1, 32), lambda i: (0, 0)),
        ],
        out_specs=pl.BlockSpec((2, BLK, 32), lambda i: (0, i, 0)),
        out_shape=jax.ShapeDtypeStruct((2, NP, 32), jnp.bfloat16),
    )(z2.reshape(2, NP, 32), y2, dinv_nm, w3, b2.reshape(1, 32))


def _pool_body(z_ref, y_ref, dinv_ref, b_ref, batch_ref, sums_ref, cnts_ref):
    i = pl.program_id(0)
    z = jnp.concatenate([z_ref[0], z_ref[1]], axis=1).astype(jnp.float32)
    y = jnp.concatenate([y_ref[0], y_ref[1]], axis=1).astype(jnp.float32)
    dv = dinv_ref[...]
    h = jnp.maximum(dv * (z + y) + b_ref[...], 0.0)
    bt = batch_ref[...]
    io = lax.broadcasted_iota(jnp.int32, (BLK, G), 1)
    oh = (bt == io).astype(jnp.float32)
    dn = (((0,), (0,)), ((), ()))
    ps = lax.dot_general(oh, h, dn, preferred_element_type=jnp.float32)
    pc = lax.dot_general(oh, jnp.ones((BLK, 1), jnp.float32), dn,
                         preferred_element_type=jnp.float32)

    @pl.when(i == 0)
    def _():
        sums_ref[...] = ps
        cnts_ref[...] = pc

    @pl.when(i != 0)
    def _():
        sums_ref[...] += ps
        cnts_ref[...] += pc


def _tc_pool(z3, y3, dinv_nm, b3, batch2):
    return pl.pallas_call(
        _pool_body,
        grid=(NBLK,),
        in_specs=[
            pl.BlockSpec((2, BLK, 32), lambda i: (0, i, 0)),
            pl.BlockSpec((2, BLK, 32), lambda i: (0, i, 0)),
            pl.BlockSpec((BLK, 1), lambda i: (i, 0)),
            pl.BlockSpec((1, 64), lambda i: (0, 0)),
            pl.BlockSpec((BLK, 1), lambda i: (i, 0)),
        ],
        out_specs=[
            pl.BlockSpec((G, 64), lambda i: (0, 0)),
            pl.BlockSpec((G, 1), lambda i: (0, 0)),
        ],
        out_shape=[
            jax.ShapeDtypeStruct((G, 64), jnp.float32),
            jax.ShapeDtypeStruct((G, 1), jnp.float32),
        ],
    )(z3.reshape(2, NP, 32), y3, dinv_nm, b3.reshape(1, 64), batch2)


def _head_body(sums_ref, cnts_ref, wfc_ref, bfc_ref, out_ref):
    pooled = sums_ref[...] / jnp.maximum(cnts_ref[...], 1.0)
    logits = jnp.dot(pooled, wfc_ref[...],
                     preferred_element_type=jnp.float32) + bfc_ref[...]
    m = jnp.max(logits, axis=1, keepdims=True)
    e = jnp.exp(logits - m)
    lse = jnp.log(jnp.sum(e, axis=1, keepdims=True)) + m
    out_ref[...] = logits - lse


def _tc_head(sums, cnts, wfc, bfc):
    return pl.pallas_call(
        _head_body,
        out_shape=jax.ShapeDtypeStruct((G, 2), jnp.float32),
    )(sums, cnts, wfc, bfc.reshape(1, 2))


def kernel(x, edge_index, batch, W1, b1, W2, b2, W3, b3, Wfc, bfc):
    src = edge_index[0]
    dst = edge_index[1]
    pad = EPAD - E
    srcp = jnp.concatenate(
        [src, jnp.zeros((pad,), jnp.int32)]).reshape(ESUB, SUB)
    dstp = jnp.concatenate(
        [dst, N + (jnp.arange(pad, dtype=jnp.int32) % NSINK)]
    ).reshape(ESUB, SUB)
    zeros16 = jnp.zeros((NP, LANES), jnp.float32)
    zeros32b = jnp.zeros((NP, 32), jnp.bfloat16)
    ones16 = jnp.ones((SUB, LANES), jnp.float32)

    xq = jnp.pad(x, ((0, NP - N), (0, 0))).reshape(NPF, 40)
    batchpad = jnp.pad(batch.reshape(N, 1), ((0, NP - N), (0, 0)),
                       constant_values=G)
    eye8 = jnp.eye(8, dtype=jnp.float32)
    w1big = jnp.kron(eye8, W1)
    w2big = jnp.kron(eye8, W2)
    b1t = jnp.tile(b1, 8).reshape(1, 128)
    s256 = jnp.zeros((128, 256), jnp.float32).at[jnp.arange(8) * 16].set(
        jnp.kron(eye8, jnp.ones((1, 32), jnp.float32)))

    hist = _get_deg_kernel()(ones16, dstp, zeros16).reshape(2, NPF, 128)
    dinv128, y1p = _tc0(hist, xq, w1big)
    dinv_nm = dinv128.reshape(NP, LANES)[:, :1]
    z1 = _make_agg(1, True)(y1p.reshape(NP, LANES), srcp, dstp, zeros16)
    y2 = _tc_mid12(z1.reshape(2, NPF, 128), y1p, dinv128, w2big, b1t,
                   s256).reshape(NP, 32)
    z2 = _make_agg(1, True, ncols=32, dtype=jnp.bfloat16, uch=10)(
        y2, srcp, dstp, zeros32b)
    y3 = _tc_mid23(z2, y2, dinv_nm, W3, b2)
    z3 = _make_agg(2, False, ncols=32, dtype=jnp.bfloat16, uch=10)(
        y3.reshape(2 * NP, 32), srcp, dstp, zeros32b)
    sums, cnts = _tc_pool(z3, y3, dinv_nm, b3, batchpad)
    return _tc_head(sums, cnts, Wfc, bfc)
